# Initial kernel scaffold; baseline (speedup 1.0000x reference)
import jax
import jax.numpy as jnp
from jax import lax
from jax.experimental import pallas as pl
from jax.experimental.pallas import tpu as pltpu
from jax.experimental.pallas import tpu_sc as plsc

N = 10000
E = 320000
G = 64
F1 = 128
FP = 16
NC, NS = 2, 16
NW = NC * NS
CHUNK = 128
EPW = -(-E // (NW * CHUNK)) * CHUNK
E_PAD = EPW * NW
NCHUNK = EPW // CHUNK
RPT = N // NS
NPAD = N + 16


def _sc_mesh():
    return plsc.VectorSubcoreMesh(core_axis_name="c", subcore_axis_name="s")


def _deg_body(dst_hbm, ones_hbm, zeros_hbm, out_hbm, acc_sh, dst_v, rows_v, sem):
    c = lax.axis_index("c")
    s = lax.axis_index("s")
    w = s * NC + c
    pltpu.sync_copy(zeros_hbm, acc_sh.at[pl.ds(s * RPT, RPT)])
    pltpu.sync_copy(ones_hbm, rows_v)
    plsc.subcore_barrier()
    base = w * EPW

    @pl.loop(0, NCHUNK)
    def _chunk(i):
        pltpu.sync_copy(dst_hbm.at[pl.ds(base + i * CHUNK, CHUNK)], dst_v)
        pltpu.sync_copy(rows_v, acc_sh.at[dst_v], add=True)

    plsc.subcore_barrier()
    pltpu.sync_copy(acc_sh.at[pl.ds(s * RPT, RPT)],
                    out_hbm.at[c, pl.ds(s * RPT, RPT)])


def _sc_degree(dst_pad, ones_blk, zeros_blk):
    k = pl.kernel(
        _deg_body,
        out_type=jax.ShapeDtypeStruct((NC, N, FP), jnp.float32),
        mesh=_sc_mesh(),
        scratch_types=[
            pltpu.VMEM_SHARED((NPAD, FP), jnp.float32),
            pltpu.VMEM((CHUNK,), jnp.int32),
            pltpu.VMEM((CHUNK, FP), jnp.float32),
            pltpu.SemaphoreType.DMA,
        ],
    )
    return k(dst_pad, ones_blk, zeros_blk)


def _make_prop_body(feat):
    def body(src_hbm, dst_hbm, table_hbm, zeros_hbm, out_hbm,
             acc_sh, src_v, dst_v, rows_v, sem):
        c = lax.axis_index("c")
        s = lax.axis_index("s")
        w = s * NC + c
        pltpu.sync_copy(zeros_hbm, acc_sh.at[pl.ds(s * RPT, RPT)])
        plsc.subcore_barrier()
        base = w * EPW

        @pl.loop(0, NCHUNK)
        def _chunk(i):
            off = base + i * CHUNK
            pltpu.sync_copy(src_hbm.at[pl.ds(off, CHUNK)], src_v)
            pltpu.sync_copy(dst_hbm.at[pl.ds(off, CHUNK)], dst_v)
            pltpu.async_copy(table_hbm.at[src_v], rows_v, sem).wait()
            pltpu.sync_copy(rows_v, acc_sh.at[dst_v], add=True)

        plsc.subcore_barrier()
        pltpu.sync_copy(acc_sh.at[pl.ds(s * RPT, RPT)],
                        out_hbm.at[c, pl.ds(s * RPT, RPT)])

    return body


def _sc_propagate(src_pad, dst_pad, table, zeros_blk, feat):
    k = pl.kernel(
        _make_prop_body(feat),
        out_type=jax.ShapeDtypeStruct((NC, N, feat), jnp.float32),
        mesh=_sc_mesh(),
        scratch_types=[
            pltpu.VMEM_SHARED((NPAD, feat), jnp.float32),
            pltpu.VMEM((CHUNK,), jnp.int32),
            pltpu.VMEM((CHUNK,), jnp.int32),
            pltpu.VMEM((CHUNK, feat), jnp.float32),
            pltpu.SemaphoreType.DMA,
        ],
    )
    return k(src_pad, dst_pad, table, zeros_blk)


def _tc_a_body(deg_ref, x_ref, w1_ref, y_ref, dinv_ref):
    p0 = deg_ref[0, :, 0:1]
    p1 = deg_ref[1, :, 0:1]
    dinv = lax.rsqrt(1.0 + p0 + p1)
    xw = jnp.dot(x_ref[...], w1_ref[...], preferred_element_type=jnp.float32)
    y_ref[0:N, :] = xw * dinv
    y_ref[N:NPAD, :] = jnp.zeros((NPAD - N, F1), jnp.float32)
    dinv_ref[...] = dinv


def _tc_a(deg_parts, x, w1):
    return pl.pallas_call(
        _tc_a_body,
        out_shape=[
            jax.ShapeDtypeStruct((NPAD, F1), jnp.float32),
            jax.ShapeDtypeStruct((N, 1), jnp.float32),
        ],
    )(deg_parts, x, w1)


def _tc_b_body(z_ref, y_ref, dinv_ref, b1_ref, w2_ref, u_ref):
    dinv = dinv_ref[...]
    h = dinv * (z_ref[0] + z_ref[1] + y_ref[0:N, :]) + b1_ref[...]
    h = jnp.where(h >= 0.0, h, 0.01 * h)
    u = jnp.dot(h, w2_ref[...], preferred_element_type=jnp.float32) * dinv
    u_ref[0:N, :] = u
    u_ref[N:NPAD, :] = jnp.zeros((NPAD - N, FP), jnp.float32)


def _tc_b(z_parts, y_pad, dinv, b1, w2p):
    return pl.pallas_call(
        _tc_b_body,
        out_shape=jax.ShapeDtypeStruct((NPAD, FP), jnp.float32),
    )(z_parts, y_pad, dinv, b1, w2p)


def _tc_c_body(z2_ref, u_ref, dinv_ref, b2_ref, batch_ref, out_ref):
    o = dinv_ref[...] * (z2_ref[0] + z2_ref[1] + u_ref[0:N, :]) + b2_ref[...]
    col = lax.broadcasted_iota(jnp.int32, (N, FP), 1)
    feat = jnp.where(col == 4, 1.0, o)
    onehot = (batch_ref[...] ==
              lax.broadcasted_iota(jnp.int32, (N, G), 1)).astype(jnp.float32)
    pooled = lax.dot_general(onehot, feat, (((0,), (0,)), ((), ())),
                             preferred_element_type=jnp.float32)
    cnt = jnp.maximum(pooled[:, 4:5], 1.0)
    mean = pooled[:, 0:4] / cnt
    m = jnp.max(mean, axis=1, keepdims=True)
    e = jnp.exp(mean - m)
    out_ref[...] = e / jnp.sum(e, axis=1, keepdims=True)


def _tc_c(z2_parts, u_pad, dinv, b2p, batch2d):
    return pl.pallas_call(
        _tc_c_body,
        out_shape=jax.ShapeDtypeStruct((G, 4), jnp.float32),
    )(z2_parts, u_pad, dinv, b2p, batch2d)


def kernel(x, edge_index, batch, W1, b1, W2, b2):
    ei = edge_index.astype(jnp.int32)
    pad = jnp.full((E_PAD - E,), N, dtype=jnp.int32)
    src_pad = jnp.concatenate([ei[0], pad])
    dst_pad = jnp.concatenate([ei[1], pad])
    batch2d = batch.astype(jnp.int32).reshape(N, 1)
    b1r = b1.reshape(1, F1)
    w2p = jnp.zeros((F1, FP), jnp.float32).at[:, 0:4].set(W2)
    b2p = jnp.zeros((1, FP), jnp.float32).at[0, 0:4].set(b2)
    ones_blk = jnp.ones((CHUNK, FP), jnp.float32)
    zeros_fp = jnp.zeros((RPT, FP), jnp.float32)
    zeros_f1 = jnp.zeros((RPT, F1), jnp.float32)

    deg_parts = _sc_degree(dst_pad, ones_blk, zeros_fp)
    y_pad, dinv = _tc_a(deg_parts, x, W1)
    z_parts = _sc_propagate(src_pad, dst_pad, y_pad, zeros_f1, F1)
    u_pad = _tc_b(z_parts, y_pad, dinv, b1r, w2p)
    z2_parts = _sc_propagate(src_pad, dst_pad, u_pad, zeros_fp, FP)
    return _tc_c(z2_parts, u_pad, dinv, b2p, batch2d)


# speedup vs baseline: 14.9630x; 14.9630x over previous
import jax
import jax.numpy as jnp
from jax import lax
from jax.experimental import pallas as pl
from jax.experimental.pallas import tpu as pltpu
from jax.experimental.pallas import tpu_sc as plsc

N = 10000
E = 320000
G = 64
F1 = 128
FP = 16
NC, NS = 2, 16
NW = NC * NS
CHUNK = 128
EPW = -(-E // (NW * CHUNK)) * CHUNK
E_PAD = EPW * NW
NCHUNK = EPW // CHUNK
NPAD = 10112
RPT = NPAD // NS


def _sc_mesh():
    return plsc.VectorSubcoreMesh(core_axis_name="c", subcore_axis_name="s")


def _deg_body(dst_hbm, ones_hbm, zeros_hbm, out_hbm, acc_sh, dst_v, rows_v, sem):
    c = lax.axis_index("c")
    s = lax.axis_index("s")
    w = s * NC + c
    pltpu.sync_copy(zeros_hbm, acc_sh.at[pl.ds(s * RPT, RPT)])
    pltpu.sync_copy(ones_hbm, rows_v)
    plsc.subcore_barrier()
    base = w * EPW

    @pl.loop(0, NCHUNK)
    def _chunk(i):
        pltpu.sync_copy(dst_hbm.at[pl.ds(base + i * CHUNK, CHUNK)], dst_v)
        pltpu.sync_copy(rows_v, acc_sh.at[dst_v], add=True)

    plsc.subcore_barrier()
    pltpu.sync_copy(acc_sh.at[pl.ds(s * RPT, RPT)],
                    out_hbm.at[c, pl.ds(s * RPT, RPT)])


def _sc_degree(dst_pad, ones_blk, zeros_blk):
    k = pl.kernel(
        _deg_body,
        out_type=jax.ShapeDtypeStruct((NC, NPAD, FP), jnp.float32),
        mesh=_sc_mesh(),
        compiler_params=pltpu.CompilerParams(use_tc_tiling_on_sc=False),
        scratch_types=[
            pltpu.VMEM_SHARED((NPAD, FP), jnp.float32),
            pltpu.VMEM((CHUNK,), jnp.int32),
            pltpu.VMEM((CHUNK, FP), jnp.float32),
            pltpu.SemaphoreType.DMA,
        ],
    )
    return k(dst_pad, ones_blk, zeros_blk)


def _make_prop_body(feat):
    def body(src_hbm, dst_hbm, table_hbm, zeros_hbm, out_hbm,
             acc_sh, src_v, dst_v, rows_v, sem):
        c = lax.axis_index("c")
        s = lax.axis_index("s")
        w = s * NC + c
        pltpu.sync_copy(zeros_hbm, acc_sh.at[pl.ds(s * RPT, RPT)])
        plsc.subcore_barrier()
        base = w * EPW

        @pl.loop(0, NCHUNK)
        def _chunk(i):
            off = base + i * CHUNK
            pltpu.sync_copy(src_hbm.at[pl.ds(off, CHUNK)], src_v)
            pltpu.sync_copy(dst_hbm.at[pl.ds(off, CHUNK)], dst_v)
            pltpu.async_copy(table_hbm.at[src_v], rows_v, sem).wait()
            pltpu.sync_copy(rows_v, acc_sh.at[dst_v], add=True)

        plsc.subcore_barrier()
        pltpu.sync_copy(acc_sh.at[pl.ds(s * RPT, RPT)],
                        out_hbm.at[c, pl.ds(s * RPT, RPT)])

    return body


def _sc_propagate(src_pad, dst_pad, table, zeros_blk, feat):
    k = pl.kernel(
        _make_prop_body(feat),
        out_type=jax.ShapeDtypeStruct((NC, NPAD, feat), jnp.float32),
        mesh=_sc_mesh(),
        compiler_params=pltpu.CompilerParams(use_tc_tiling_on_sc=(feat == F1)),
        scratch_types=[
            pltpu.VMEM_SHARED((NPAD, feat), jnp.float32),
            pltpu.VMEM((CHUNK,), jnp.int32),
            pltpu.VMEM((CHUNK,), jnp.int32),
            pltpu.VMEM((CHUNK, feat), jnp.float32),
            pltpu.SemaphoreType.DMA,
        ],
    )
    return k(src_pad, dst_pad, table, zeros_blk)


def _tc_a_body(deg_ref, x_ref, w1_ref, y_ref, dinv_ref):
    p0 = deg_ref[0, 0:N, 0:1]
    p1 = deg_ref[1, 0:N, 0:1]
    dinv = lax.rsqrt(1.0 + p0 + p1)
    xw = jnp.dot(x_ref[...], w1_ref[...], preferred_element_type=jnp.float32)
    y_ref[0:N, :] = xw * dinv
    y_ref[N:NPAD, :] = jnp.zeros((NPAD - N, F1), jnp.float32)
    dinv_ref[...] = dinv


def _tc_a(deg_parts, x, w1):
    return pl.pallas_call(
        _tc_a_body,
        out_shape=[
            jax.ShapeDtypeStruct((NPAD, F1), jnp.float32),
            jax.ShapeDtypeStruct((N, 1), jnp.float32),
        ],
    )(deg_parts, x, w1)


def _tc_b_body(z_ref, y_ref, dinv_ref, b1_ref, w2_ref, u_ref):
    dinv = dinv_ref[...]
    h = dinv * (z_ref[0, 0:N, :] + z_ref[1, 0:N, :] + y_ref[0:N, :]) + b1_ref[...]
    h = jnp.where(h >= 0.0, h, 0.01 * h)
    u = jnp.dot(h, w2_ref[...], preferred_element_type=jnp.float32) * dinv
    u_ref[0:N, :] = u
    u_ref[N:NPAD, :] = jnp.zeros((NPAD - N, FP), jnp.float32)


def _tc_b(z_parts, y_pad, dinv, b1, w2p):
    return pl.pallas_call(
        _tc_b_body,
        out_shape=jax.ShapeDtypeStruct((NPAD, FP), jnp.float32),
    )(z_parts, y_pad, dinv, b1, w2p)


def _tc_c_body(z2_ref, u_ref, dinv_ref, b2_ref, batch_ref, out_ref):
    o = dinv_ref[...] * (z2_ref[0, 0:N, :] + z2_ref[1, 0:N, :] + u_ref[0:N, :]) + b2_ref[...]
    col = lax.broadcasted_iota(jnp.int32, (N, FP), 1)
    feat = jnp.where(col == 4, 1.0, o)
    onehot = (batch_ref[...] ==
              lax.broadcasted_iota(jnp.int32, (N, G), 1)).astype(jnp.float32)
    pooled = lax.dot_general(onehot, feat, (((0,), (0,)), ((), ())),
                             preferred_element_type=jnp.float32)
    cnt = jnp.maximum(pooled[:, 4:5], 1.0)
    mean = pooled[:, 0:4] / cnt
    m = jnp.max(mean, axis=1, keepdims=True)
    e = jnp.exp(mean - m)
    out_ref[...] = e / jnp.sum(e, axis=1, keepdims=True)


def _tc_c(z2_parts, u_pad, dinv, b2p, batch2d):
    return pl.pallas_call(
        _tc_c_body,
        out_shape=jax.ShapeDtypeStruct((G, 4), jnp.float32),
    )(z2_parts, u_pad, dinv, b2p, batch2d)


def kernel(x, edge_index, batch, W1, b1, W2, b2):
    ei = edge_index.astype(jnp.int32)
    pad = jnp.full((E_PAD - E,), N, dtype=jnp.int32)
    src_pad = jnp.concatenate([ei[0], pad])
    dst_pad = jnp.concatenate([ei[1], pad])
    batch2d = batch.astype(jnp.int32).reshape(N, 1)
    b1r = b1.reshape(1, F1)
    w2p = jnp.zeros((F1, FP), jnp.float32).at[:, 0:4].set(W2)
    b2p = jnp.zeros((1, FP), jnp.float32).at[0, 0:4].set(b2)
    ones_blk = jnp.ones((CHUNK, FP), jnp.float32)
    zeros_fp = jnp.zeros((RPT, FP), jnp.float32)
    zeros_f1 = jnp.zeros((RPT, F1), jnp.float32)

    deg_parts = _sc_degree(dst_pad, ones_blk, zeros_fp)
    y_pad, dinv = _tc_a(deg_parts, x, W1)
    z_parts = _sc_propagate(src_pad, dst_pad, y_pad, zeros_f1, F1)
    u_pad = _tc_b(z_parts, y_pad, dinv, b1r, w2p)
    z2_parts = _sc_propagate(src_pad, dst_pad, u_pad, zeros_fp, FP)
    return _tc_c(z2_parts, u_pad, dinv, b2p, batch2d)


# speedup vs baseline: 19.4948x; 1.3029x over previous
import jax
import jax.numpy as jnp
from jax import lax
from jax.experimental import pallas as pl
from jax.experimental.pallas import tpu as pltpu
from jax.experimental.pallas import tpu_sc as plsc

N = 10000
E = 320000
G = 64
F1 = 128
FP = 16
NC, NS = 2, 16
NW = NC * NS
CHUNK = 128
NCHUNK = 80
EPW = NCHUNK * CHUNK
E_PAD = EPW * NW
FH = F1 // NC
NCHUNK_F = E_PAD // (NS * CHUNK)
NPAD = 10112
RPT = NPAD // NS


def _sc_mesh():
    return plsc.VectorSubcoreMesh(core_axis_name="c", subcore_axis_name="s")


def _deg_body(dst_hbm, ones_hbm, zeros_hbm, out_hbm, acc_sh, idx2_v, rows_v, sem):
    c = lax.axis_index("c")
    s = lax.axis_index("s")
    w = s * NC + c
    pltpu.sync_copy(zeros_hbm, acc_sh.at[pl.ds(s * RPT, RPT)])
    pltpu.sync_copy(ones_hbm, rows_v)
    pltpu.sync_copy(dst_hbm.at[w], idx2_v)
    plsc.subcore_barrier()

    @pl.loop(0, NCHUNK)
    def _fire(i):
        pltpu.async_copy(rows_v, acc_sh.at[idx2_v.at[i]], sem, add=True)

    @pl.loop(0, NCHUNK)
    def _drain(i):
        pltpu.make_async_copy(rows_v, acc_sh.at[idx2_v.at[i]], sem).wait()

    plsc.subcore_barrier()
    pltpu.sync_copy(acc_sh.at[pl.ds(s * RPT, RPT)],
                    out_hbm.at[c, pl.ds(s * RPT, RPT)])


def _sc_degree(dst3, ones_blk, zeros_blk):
    k = pl.kernel(
        _deg_body,
        out_type=jax.ShapeDtypeStruct((NC, NPAD, FP), jnp.float32),
        mesh=_sc_mesh(),
        compiler_params=pltpu.CompilerParams(use_tc_tiling_on_sc=False),
        scratch_types=[
            pltpu.VMEM_SHARED((NPAD, FP), jnp.float32),
            pltpu.VMEM((NCHUNK, CHUNK), jnp.int32),
            pltpu.VMEM((CHUNK, FP), jnp.float32),
            pltpu.SemaphoreType.DMA,
        ],
    )
    return k(dst3, ones_blk, zeros_blk)


def _make_prop_body(featc, ring, ahead, nchunks, fsplit):
    def body(src_hbm, dst_hbm, table_hbm, zeros_hbm, out_hbm,
             acc_sh, src2_v, dst2_v, *rest):
        rows = rest[:ring]
        sg = rest[ring:2 * ring]
        ss = rest[2 * ring:3 * ring]
        c = lax.axis_index("c")
        s = lax.axis_index("s")
        slab = s if fsplit else s * NC + c
        table = table_hbm.at[c] if fsplit else table_hbm
        pltpu.sync_copy(zeros_hbm, acc_sh.at[pl.ds(s * RPT, RPT)])
        pltpu.sync_copy(src_hbm.at[slab], src2_v)
        pltpu.sync_copy(dst_hbm.at[slab], dst2_v)
        plsc.subcore_barrier()

        for g in range(ahead):
            pltpu.async_copy(table.at[src2_v.at[g]], rows[g], sg[g])

        @pl.loop(0, nchunks // ring)
        def _blk(j):
            for b in range(ring):
                g = j * ring + b
                pltpu.make_async_copy(table.at[src2_v.at[g]],
                                      rows[b], sg[b]).wait()
                pltpu.async_copy(rows[b], acc_sh.at[dst2_v.at[g]], ss[b],
                                 add=True)
                g2 = g + ahead
                b2 = (b + ahead) % ring

                @pl.when(g2 < nchunks)
                def _fire_next():
                    @pl.when(g2 >= ring)
                    def _free_buf():
                        pltpu.make_async_copy(
                            rows[b2], acc_sh.at[dst2_v.at[g2 - ring]],
                            ss[b2]).wait()

                    pltpu.async_copy(table.at[src2_v.at[g2]],
                                     rows[b2], sg[b2])

        for b in range(ring):
            g = nchunks - ring + b
            pltpu.make_async_copy(rows[b], acc_sh.at[dst2_v.at[g]],
                                  ss[b]).wait()

        plsc.subcore_barrier()
        pltpu.sync_copy(acc_sh.at[pl.ds(s * RPT, RPT)],
                        out_hbm.at[c, pl.ds(s * RPT, RPT)])

    return body


def _sc_propagate(src3, dst3, table, zeros_blk, featc, ring, ahead, nchunks,
                  fsplit):
    k = pl.kernel(
        _make_prop_body(featc, ring, ahead, nchunks, fsplit),
        out_type=jax.ShapeDtypeStruct((NC, NPAD, featc), jnp.float32),
        mesh=_sc_mesh(),
        compiler_params=pltpu.CompilerParams(use_tc_tiling_on_sc=False),
        scratch_types=(
            [
                pltpu.VMEM_SHARED((NPAD, featc), jnp.float32),
                pltpu.VMEM((nchunks, CHUNK), jnp.int32),
                pltpu.VMEM((nchunks, CHUNK), jnp.int32),
            ]
            + [pltpu.VMEM((CHUNK, featc), jnp.float32) for _ in range(ring)]
            + [pltpu.SemaphoreType.DMA for _ in range(2 * ring)]
        ),
    )
    return k(src3, dst3, table, zeros_blk)


def _tc_a_body(deg_ref, x_ref, w1_ref, y_ref, dinv_ref):
    p0 = deg_ref[0, 0:N, 0:1]
    p1 = deg_ref[1, 0:N, 0:1]
    dinv = lax.rsqrt(1.0 + p0 + p1)
    xw = jnp.dot(x_ref[...], w1_ref[...],
                 preferred_element_type=jnp.float32) * dinv
    y_ref[0, 0:N, :] = xw[:, 0:FH]
    y_ref[1, 0:N, :] = xw[:, FH:F1]
    y_ref[0, N:NPAD, :] = jnp.zeros((NPAD - N, FH), jnp.float32)
    y_ref[1, N:NPAD, :] = jnp.zeros((NPAD - N, FH), jnp.float32)
    dinv_ref[...] = dinv


def _tc_a(deg_parts, x, w1):
    return pl.pallas_call(
        _tc_a_body,
        out_shape=[
            jax.ShapeDtypeStruct((NC, NPAD, FH), jnp.float32),
            jax.ShapeDtypeStruct((N, 1), jnp.float32),
        ],
    )(deg_parts, x, w1)


def _tc_b_body(z_ref, y_ref, dinv_ref, b1_ref, w2_ref, u_ref):
    dinv = dinv_ref[...]
    zy = jnp.concatenate(
        [z_ref[0, 0:N, :] + y_ref[0, 0:N, :],
         z_ref[1, 0:N, :] + y_ref[1, 0:N, :]], axis=1)
    h = dinv * zy + b1_ref[...]
    h = jnp.where(h >= 0.0, h, 0.01 * h)
    u = jnp.dot(h, w2_ref[...], preferred_element_type=jnp.float32) * dinv
    u_ref[0:N, :] = u
    u_ref[N:NPAD, :] = jnp.zeros((NPAD - N, FP), jnp.float32)


def _tc_b(z_parts, y_pad, dinv, b1, w2p):
    return pl.pallas_call(
        _tc_b_body,
        out_shape=jax.ShapeDtypeStruct((NPAD, FP), jnp.float32),
    )(z_parts, y_pad, dinv, b1, w2p)


def _tc_c_body(z2_ref, u_ref, dinv_ref, b2_ref, batch_ref, out_ref):
    o = dinv_ref[...] * (z2_ref[0, 0:N, :] + z2_ref[1, 0:N, :] + u_ref[0:N, :]) + b2_ref[...]
    col = lax.broadcasted_iota(jnp.int32, (N, FP), 1)
    feat = jnp.where(col == 4, 1.0, o)
    onehot = (batch_ref[...] ==
              lax.broadcasted_iota(jnp.int32, (N, G), 1)).astype(jnp.float32)
    pooled = lax.dot_general(onehot, feat, (((0,), (0,)), ((), ())),
                             preferred_element_type=jnp.float32)
    cnt = jnp.maximum(pooled[:, 4:5], 1.0)
    mean = pooled[:, 0:4] / cnt
    m = jnp.max(mean, axis=1, keepdims=True)
    e = jnp.exp(mean - m)
    out_ref[...] = e / jnp.sum(e, axis=1, keepdims=True)


def _tc_c(z2_parts, u_pad, dinv, b2p, batch2d):
    return pl.pallas_call(
        _tc_c_body,
        out_shape=jax.ShapeDtypeStruct((G, 4), jnp.float32),
    )(z2_parts, u_pad, dinv, b2p, batch2d)


def kernel(x, edge_index, batch, W1, b1, W2, b2):
    ei = edge_index.astype(jnp.int32)
    pad = jnp.full((E_PAD - E,), N, dtype=jnp.int32)
    src_flat = jnp.concatenate([ei[0], pad])
    dst_flat = jnp.concatenate([ei[1], pad])
    src_pad = src_flat.reshape(NW, NCHUNK, CHUNK)
    dst_pad = dst_flat.reshape(NW, NCHUNK, CHUNK)
    src_f = src_flat.reshape(NS, NCHUNK_F, CHUNK)
    dst_f = dst_flat.reshape(NS, NCHUNK_F, CHUNK)
    batch2d = batch.astype(jnp.int32).reshape(N, 1)
    b1r = b1.reshape(1, F1)
    w2p = jnp.zeros((F1, FP), jnp.float32).at[:, 0:4].set(W2)
    b2p = jnp.zeros((1, FP), jnp.float32).at[0, 0:4].set(b2)
    ones_blk = jnp.ones((CHUNK, FP), jnp.float32)
    zeros_fp = jnp.zeros((RPT, FP), jnp.float32)
    zeros_fh = jnp.zeros((RPT, FH), jnp.float32)

    deg_parts = _sc_degree(dst_pad, ones_blk, zeros_fp)
    y2, dinv = _tc_a(deg_parts, x, W1)
    z_parts = _sc_propagate(src_f, dst_f, y2, zeros_fh, FH, 4, 2, NCHUNK_F,
                            True)
    u_pad = _tc_b(z_parts, y2, dinv, b1r, w2p)
    z2_parts = _sc_propagate(src_pad, dst_pad, u_pad, zeros_fp, FP, 8, 4,
                             NCHUNK, False)
    return _tc_c(z2_parts, u_pad, dinv, b2p, batch2d)


# speedup vs baseline: 19.8688x; 1.0192x over previous
import jax
import jax.numpy as jnp
from jax import lax
from jax.experimental import pallas as pl
from jax.experimental.pallas import tpu as pltpu
from jax.experimental.pallas import tpu_sc as plsc

N = 10000
E = 320000
G = 64
F1 = 128
FP = 16
NC, NS = 2, 16
NW = NC * NS
CHUNK = 128
NCHUNK = 80
EPW = NCHUNK * CHUNK
E_PAD = EPW * NW
FH = F1 // NC
NCHUNK_F = E_PAD // (NS * CHUNK)
NPAD = 10112
RPT = NPAD // NS


def _sc_mesh():
    return plsc.VectorSubcoreMesh(core_axis_name="c", subcore_axis_name="s")


def _deg_body(dst_hbm, ones_hbm, zeros_hbm, out_hbm, acc_sh, idx2_v, rows_v, sem):
    c = lax.axis_index("c")
    s = lax.axis_index("s")
    w = s * NC + c
    pltpu.sync_copy(zeros_hbm, acc_sh.at[pl.ds(s * RPT, RPT)])
    pltpu.sync_copy(ones_hbm, rows_v)
    pltpu.sync_copy(dst_hbm.at[w], idx2_v)
    plsc.subcore_barrier()

    @pl.loop(0, NCHUNK)
    def _fire(i):
        pltpu.async_copy(rows_v, acc_sh.at[idx2_v.at[i]], sem, add=True)

    @pl.loop(0, NCHUNK)
    def _drain(i):
        pltpu.make_async_copy(rows_v, acc_sh.at[idx2_v.at[i]], sem).wait()

    plsc.subcore_barrier()
    pltpu.sync_copy(acc_sh.at[pl.ds(s * RPT, RPT)],
                    out_hbm.at[c, pl.ds(s * RPT, RPT)])


def _sc_degree(dst3, ones_blk, zeros_blk):
    k = pl.kernel(
        _deg_body,
        out_type=jax.ShapeDtypeStruct((NC, NPAD, FP), jnp.float32),
        mesh=_sc_mesh(),
        compiler_params=pltpu.CompilerParams(use_tc_tiling_on_sc=False),
        scratch_types=[
            pltpu.VMEM_SHARED((NPAD, FP), jnp.float32),
            pltpu.VMEM((NCHUNK, CHUNK), jnp.int32),
            pltpu.VMEM((CHUNK, FP), jnp.float32),
            pltpu.SemaphoreType.DMA,
        ],
    )
    return k(dst3, ones_blk, zeros_blk)


def _make_prop_body(featc, ring, ahead, nchunks, fsplit):
    def body(src_hbm, dst_hbm, table_hbm, zeros_hbm, out_hbm,
             acc_sh, src2_v, dst2_v, *rest):
        rows = rest[:ring]
        sg = rest[ring:2 * ring]
        ss = rest[2 * ring:3 * ring]
        c = lax.axis_index("c")
        s = lax.axis_index("s")
        slab = s if fsplit else s * NC + c
        table = table_hbm.at[c] if fsplit else table_hbm
        pltpu.sync_copy(zeros_hbm, acc_sh.at[pl.ds(s * RPT, RPT)])
        pltpu.sync_copy(src_hbm.at[slab], src2_v)
        pltpu.sync_copy(dst_hbm.at[slab], dst2_v)
        plsc.subcore_barrier()

        for g in range(ahead):
            pltpu.async_copy(table.at[src2_v.at[g]], rows[g], sg[g])

        @pl.loop(0, nchunks // ring)
        def _blk(j):
            for b in range(ring):
                g = j * ring + b
                pltpu.make_async_copy(table.at[src2_v.at[g]],
                                      rows[b], sg[b]).wait()
                pltpu.async_copy(rows[b], acc_sh.at[dst2_v.at[g]], ss[b],
                                 add=True)
                g2 = g + ahead
                b2 = (b + ahead) % ring

                @pl.when(g2 < nchunks)
                def _fire_next():
                    @pl.when(g2 >= ring)
                    def _free_buf():
                        pltpu.make_async_copy(
                            rows[b2], acc_sh.at[dst2_v.at[g2 - ring]],
                            ss[b2]).wait()

                    pltpu.async_copy(table.at[src2_v.at[g2]],
                                     rows[b2], sg[b2])

        for b in range(ring):
            g = nchunks - ring + b
            pltpu.make_async_copy(rows[b], acc_sh.at[dst2_v.at[g]],
                                  ss[b]).wait()

        plsc.subcore_barrier()
        pltpu.sync_copy(acc_sh.at[pl.ds(s * RPT, RPT)],
                        out_hbm.at[c, pl.ds(s * RPT, RPT)])

    return body


def _sc_propagate(src3, dst3, table, zeros_blk, featc, ring, ahead, nchunks,
                  fsplit):
    k = pl.kernel(
        _make_prop_body(featc, ring, ahead, nchunks, fsplit),
        out_type=jax.ShapeDtypeStruct((NC, NPAD, featc), jnp.float32),
        mesh=_sc_mesh(),
        compiler_params=pltpu.CompilerParams(use_tc_tiling_on_sc=False),
        scratch_types=(
            [
                pltpu.VMEM_SHARED((NPAD, featc), jnp.float32),
                pltpu.VMEM((nchunks, CHUNK), jnp.int32),
                pltpu.VMEM((nchunks, CHUNK), jnp.int32),
            ]
            + [pltpu.VMEM((CHUNK, featc), jnp.float32) for _ in range(ring)]
            + [pltpu.SemaphoreType.DMA for _ in range(2 * ring)]
        ),
    )
    return k(src3, dst3, table, zeros_blk)


def _tc_a0_body(x_ref, w1_ref, xw_ref):
    xw_ref[...] = jnp.dot(x_ref[...], w1_ref[...],
                          preferred_element_type=jnp.float32)


def _tc_a0(x, w1):
    return pl.pallas_call(
        _tc_a0_body,
        out_shape=jax.ShapeDtypeStruct((N, F1), jnp.float32),
    )(x, w1)


def _tc_a_body(deg_ref, xw_ref, y_ref, dinv_ref):
    p0 = deg_ref[0, 0:N, 0:1]
    p1 = deg_ref[1, 0:N, 0:1]
    dinv = lax.rsqrt(1.0 + p0 + p1)
    xw = xw_ref[...] * dinv
    y_ref[0, 0:N, :] = xw[:, 0:FH]
    y_ref[1, 0:N, :] = xw[:, FH:F1]
    y_ref[0, N:NPAD, :] = jnp.zeros((NPAD - N, FH), jnp.float32)
    y_ref[1, N:NPAD, :] = jnp.zeros((NPAD - N, FH), jnp.float32)
    dinv_ref[...] = dinv


def _tc_a(deg_parts, xw):
    return pl.pallas_call(
        _tc_a_body,
        out_shape=[
            jax.ShapeDtypeStruct((NC, NPAD, FH), jnp.float32),
            jax.ShapeDtypeStruct((N, 1), jnp.float32),
        ],
    )(deg_parts, xw)


def _tc_b_body(z_ref, y_ref, dinv_ref, b1_ref, w2_ref, u_ref):
    dinv = dinv_ref[...]
    zy = jnp.concatenate(
        [z_ref[0, 0:N, :] + y_ref[0, 0:N, :],
         z_ref[1, 0:N, :] + y_ref[1, 0:N, :]], axis=1)
    h = dinv * zy + b1_ref[...]
    h = jnp.where(h >= 0.0, h, 0.01 * h)
    u = jnp.dot(h, w2_ref[...], preferred_element_type=jnp.float32) * dinv
    u_ref[0:N, :] = u
    u_ref[N:NPAD, :] = jnp.zeros((NPAD - N, FP), jnp.float32)


def _tc_b(z_parts, y_pad, dinv, b1, w2p):
    return pl.pallas_call(
        _tc_b_body,
        out_shape=jax.ShapeDtypeStruct((NPAD, FP), jnp.float32),
    )(z_parts, y_pad, dinv, b1, w2p)


def _tc_c_body(z2_ref, u_ref, dinv_ref, b2_ref, batch_ref, out_ref):
    o = dinv_ref[...] * (z2_ref[0, 0:N, :] + z2_ref[1, 0:N, :] + u_ref[0:N, :]) + b2_ref[...]
    col = lax.broadcasted_iota(jnp.int32, (N, FP), 1)
    feat = jnp.where(col == 4, 1.0, o)
    onehot = (batch_ref[...] ==
              lax.broadcasted_iota(jnp.int32, (N, G), 1)).astype(jnp.float32)
    pooled = lax.dot_general(onehot, feat, (((0,), (0,)), ((), ())),
                             preferred_element_type=jnp.float32)
    cnt = jnp.maximum(pooled[:, 4:5], 1.0)
    mean = pooled[:, 0:4] / cnt
    m = jnp.max(mean, axis=1, keepdims=True)
    e = jnp.exp(mean - m)
    out_ref[...] = e / jnp.sum(e, axis=1, keepdims=True)


def _tc_c(z2_parts, u_pad, dinv, b2p, batch2d):
    return pl.pallas_call(
        _tc_c_body,
        out_shape=jax.ShapeDtypeStruct((G, 4), jnp.float32),
    )(z2_parts, u_pad, dinv, b2p, batch2d)


def kernel(x, edge_index, batch, W1, b1, W2, b2):
    ei = edge_index.astype(jnp.int32)
    pad = jnp.full((E_PAD - E,), N, dtype=jnp.int32)
    src_flat = jnp.concatenate([ei[0], pad])
    dst_flat = jnp.concatenate([ei[1], pad])
    src_pad = src_flat.reshape(NW, NCHUNK, CHUNK)
    dst_pad = dst_flat.reshape(NW, NCHUNK, CHUNK)
    src_f = src_flat.reshape(NS, NCHUNK_F, CHUNK)
    dst_f = dst_flat.reshape(NS, NCHUNK_F, CHUNK)
    batch2d = batch.astype(jnp.int32).reshape(N, 1)
    b1r = b1.reshape(1, F1)
    w2p = jnp.zeros((F1, FP), jnp.float32).at[:, 0:4].set(W2)
    b2p = jnp.zeros((1, FP), jnp.float32).at[0, 0:4].set(b2)
    ones_blk = jnp.ones((CHUNK, FP), jnp.float32)
    zeros_fp = jnp.zeros((RPT, FP), jnp.float32)
    zeros_fh = jnp.zeros((RPT, FH), jnp.float32)

    xw = _tc_a0(x, W1)
    deg_parts = _sc_degree(dst_pad, ones_blk, zeros_fp)
    y2, dinv = _tc_a(deg_parts, xw)
    z_parts = _sc_propagate(src_f, dst_f, y2, zeros_fh, FH, 5, 3, NCHUNK_F,
                            True)
    u_pad = _tc_b(z_parts, y2, dinv, b1r, w2p)
    z2_parts = _sc_propagate(src_pad, dst_pad, u_pad, zeros_fp, FP, 10, 5,
                             NCHUNK, False)
    return _tc_c(z2_parts, u_pad, dinv, b2p, batch2d)


# speedup vs baseline: 29.5764x; 1.4886x over previous
import jax
import jax.numpy as jnp
from jax import lax
from jax.experimental import pallas as pl
from jax.experimental.pallas import tpu as pltpu
from jax.experimental.pallas import tpu_sc as plsc

N = 10000
E = 320000
G = 64
F1 = 128
FP = 16
NC, NS = 2, 16
NW = NC * NS
CHUNK = 128
NCHUNK = 80
EPW = NCHUNK * CHUNK
E_PAD = EPW * NW
FH = F1 // NC
NCHUNK_F = E_PAD // (NS * CHUNK)
NPAD = 10112
RPT = NPAD // NS


def _sc_mesh():
    return plsc.VectorSubcoreMesh(core_axis_name="c", subcore_axis_name="s")


def _deg_body(dst_hbm, ones_hbm, zeros_hbm, out_hbm, acc_sh, idx2_v, rows_v, sem):
    c = lax.axis_index("c")
    s = lax.axis_index("s")
    w = s * NC + c
    pltpu.sync_copy(zeros_hbm, acc_sh.at[pl.ds(s * RPT, RPT)])
    pltpu.sync_copy(ones_hbm, rows_v)
    pltpu.sync_copy(dst_hbm.at[w], idx2_v)
    plsc.subcore_barrier()

    @pl.loop(0, NCHUNK)
    def _fire(i):
        pltpu.async_copy(rows_v, acc_sh.at[idx2_v.at[i]], sem, add=True)

    @pl.loop(0, NCHUNK)
    def _drain(i):
        pltpu.make_async_copy(rows_v, acc_sh.at[idx2_v.at[i]], sem).wait()

    plsc.subcore_barrier()
    pltpu.sync_copy(acc_sh.at[pl.ds(s * RPT, RPT)],
                    out_hbm.at[c, pl.ds(s * RPT, RPT)])


def _sc_degree(dst3, ones_blk, zeros_blk):
    k = pl.kernel(
        _deg_body,
        out_type=jax.ShapeDtypeStruct((NC, NPAD, FP), jnp.float32),
        mesh=_sc_mesh(),
        compiler_params=pltpu.CompilerParams(use_tc_tiling_on_sc=False),
        scratch_types=[
            pltpu.VMEM_SHARED((NPAD, FP), jnp.float32),
            pltpu.VMEM((NCHUNK, CHUNK), jnp.int32),
            pltpu.VMEM((CHUNK, FP), jnp.float32),
            pltpu.SemaphoreType.DMA,
        ],
    )
    return k(dst3, ones_blk, zeros_blk)


def _make_prop_body(featc, ring, ahead, nchunks, fsplit):
    def body(src_hbm, dst_hbm, table_hbm, zeros_hbm, out_hbm,
             acc_sh, src2_v, dst2_v, *rest):
        rows = rest[:ring]
        sg = rest[ring:2 * ring]
        ss = rest[2 * ring:3 * ring]
        c = lax.axis_index("c")
        s = lax.axis_index("s")
        slab = s if fsplit else s * NC + c
        table = table_hbm.at[c] if fsplit else table_hbm
        pltpu.sync_copy(zeros_hbm, acc_sh.at[pl.ds(s * RPT, RPT)])
        pltpu.sync_copy(src_hbm.at[slab], src2_v)
        pltpu.sync_copy(dst_hbm.at[slab], dst2_v)
        plsc.subcore_barrier()

        for g in range(ahead):
            pltpu.async_copy(table.at[src2_v.at[g]], rows[g], sg[g])

        @pl.loop(0, nchunks // ring)
        def _blk(j):
            for b in range(ring):
                g = j * ring + b
                pltpu.make_async_copy(table.at[src2_v.at[g]],
                                      rows[b], sg[b]).wait()
                pltpu.async_copy(rows[b], acc_sh.at[dst2_v.at[g]], ss[b],
                                 add=True)
                g2 = g + ahead
                b2 = (b + ahead) % ring

                @pl.when(g2 < nchunks)
                def _fire_next():
                    @pl.when(g2 >= ring)
                    def _free_buf():
                        pltpu.make_async_copy(
                            rows[b2], acc_sh.at[dst2_v.at[g2 - ring]],
                            ss[b2]).wait()

                    pltpu.async_copy(table.at[src2_v.at[g2]],
                                     rows[b2], sg[b2])

        for b in range(ring):
            g = nchunks - ring + b
            pltpu.make_async_copy(rows[b], acc_sh.at[dst2_v.at[g]],
                                  ss[b]).wait()

        plsc.subcore_barrier()
        pltpu.sync_copy(acc_sh.at[pl.ds(s * RPT, RPT)],
                        out_hbm.at[c, pl.ds(s * RPT, RPT)])

    return body


def _sc_propagate(src3, dst3, table, zeros_blk, featc, ring, ahead, nchunks,
                  fsplit, dtype):
    k = pl.kernel(
        _make_prop_body(featc, ring, ahead, nchunks, fsplit),
        out_type=jax.ShapeDtypeStruct((NC, NPAD, featc), dtype),
        mesh=_sc_mesh(),
        compiler_params=pltpu.CompilerParams(use_tc_tiling_on_sc=False),
        scratch_types=(
            [
                pltpu.VMEM_SHARED((NPAD, featc), dtype),
                pltpu.VMEM((nchunks, CHUNK), jnp.int32),
                pltpu.VMEM((nchunks, CHUNK), jnp.int32),
            ]
            + [pltpu.VMEM((CHUNK, featc), dtype) for _ in range(ring)]
            + [pltpu.SemaphoreType.DMA for _ in range(2 * ring)]
        ),
    )
    return k(src3, dst3, table, zeros_blk)


def _tc_a0_body(x_ref, w1_ref, xw_ref):
    xw_ref[...] = jnp.dot(x_ref[...], w1_ref[...],
                          preferred_element_type=jnp.float32)


def _tc_a0(x, w1):
    return pl.pallas_call(
        _tc_a0_body,
        out_shape=jax.ShapeDtypeStruct((N, F1), jnp.float32),
    )(x, w1)


def _tc_a_body(deg_ref, xw_ref, y_ref, dinv_ref):
    p0 = deg_ref[0, 0:N, 0:1]
    p1 = deg_ref[1, 0:N, 0:1]
    dinv = lax.rsqrt(1.0 + p0 + p1)
    xw = xw_ref[...] * dinv
    y_ref[0, 0:N, :] = xw[:, 0:FH].astype(jnp.bfloat16)
    y_ref[1, 0:N, :] = xw[:, FH:F1].astype(jnp.bfloat16)
    y_ref[0, N:NPAD, :] = jnp.zeros((NPAD - N, FH), jnp.bfloat16)
    y_ref[1, N:NPAD, :] = jnp.zeros((NPAD - N, FH), jnp.bfloat16)
    dinv_ref[...] = dinv


def _tc_a(deg_parts, xw):
    return pl.pallas_call(
        _tc_a_body,
        out_shape=[
            jax.ShapeDtypeStruct((NC, NPAD, FH), jnp.bfloat16),
            jax.ShapeDtypeStruct((N, 1), jnp.float32),
        ],
    )(deg_parts, xw)


def _tc_b_body(z_ref, y_ref, dinv_ref, b1_ref, w2_ref, u_ref):
    dinv = dinv_ref[...]
    zy = jnp.concatenate(
        [z_ref[0, 0:N, :].astype(jnp.float32) +
         y_ref[0, 0:N, :].astype(jnp.float32),
         z_ref[1, 0:N, :].astype(jnp.float32) +
         y_ref[1, 0:N, :].astype(jnp.float32)], axis=1)
    h = dinv * zy + b1_ref[...]
    h = jnp.where(h >= 0.0, h, 0.01 * h)
    u = jnp.dot(h, w2_ref[...], preferred_element_type=jnp.float32) * dinv
    u_ref[0:N, :] = u
    u_ref[N:NPAD, :] = jnp.zeros((NPAD - N, FP), jnp.float32)


def _tc_b(z_parts, y_pad, dinv, b1, w2p):
    return pl.pallas_call(
        _tc_b_body,
        out_shape=jax.ShapeDtypeStruct((NPAD, FP), jnp.float32),
    )(z_parts, y_pad, dinv, b1, w2p)


def _tc_c_body(z2_ref, u_ref, dinv_ref, b2_ref, batch_ref, out_ref):
    o = dinv_ref[...] * (z2_ref[0, 0:N, :] + z2_ref[1, 0:N, :] + u_ref[0:N, :]) + b2_ref[...]
    col = lax.broadcasted_iota(jnp.int32, (N, FP), 1)
    feat = jnp.where(col == 4, 1.0, o)
    onehot = (batch_ref[...] ==
              lax.broadcasted_iota(jnp.int32, (N, G), 1)).astype(jnp.float32)
    pooled = lax.dot_general(onehot, feat, (((0,), (0,)), ((), ())),
                             preferred_element_type=jnp.float32)
    cnt = jnp.maximum(pooled[:, 4:5], 1.0)
    mean = pooled[:, 0:4] / cnt
    m = jnp.max(mean, axis=1, keepdims=True)
    e = jnp.exp(mean - m)
    out_ref[...] = e / jnp.sum(e, axis=1, keepdims=True)


def _tc_c(z2_parts, u_pad, dinv, b2p, batch2d):
    return pl.pallas_call(
        _tc_c_body,
        out_shape=jax.ShapeDtypeStruct((G, 4), jnp.float32),
    )(z2_parts, u_pad, dinv, b2p, batch2d)


def kernel(x, edge_index, batch, W1, b1, W2, b2):
    ei = edge_index.astype(jnp.int32)
    pad = jnp.full((E_PAD - E,), N, dtype=jnp.int32)
    src_flat = jnp.concatenate([ei[0], pad])
    dst_flat = jnp.concatenate([ei[1], pad])
    src_pad = src_flat.reshape(NW, NCHUNK, CHUNK)
    dst_pad = dst_flat.reshape(NW, NCHUNK, CHUNK)
    src_f = src_flat.reshape(NS, NCHUNK_F, CHUNK)
    dst_f = dst_flat.reshape(NS, NCHUNK_F, CHUNK)
    batch2d = batch.astype(jnp.int32).reshape(N, 1)
    b1r = b1.reshape(1, F1)
    w2p = jnp.zeros((F1, FP), jnp.float32).at[:, 0:4].set(W2)
    b2p = jnp.zeros((1, FP), jnp.float32).at[0, 0:4].set(b2)
    ones_blk = jnp.ones((CHUNK, FP), jnp.float32)
    zeros_fp = jnp.zeros((RPT, FP), jnp.float32)
    zeros_fh = jnp.zeros((RPT, FH), jnp.bfloat16)

    xw = _tc_a0(x, W1)
    deg_parts = _sc_degree(dst_pad, ones_blk, zeros_fp)
    y2, dinv = _tc_a(deg_parts, xw)
    z_parts = _sc_propagate(src_f, dst_f, y2, zeros_fh, FH, 5, 3, NCHUNK_F,
                            True, jnp.bfloat16)
    u_pad = _tc_b(z_parts, y2, dinv, b1r, w2p)
    z2_parts = _sc_propagate(src_pad, dst_pad, u_pad, zeros_fp, FP, 10, 5,
                             NCHUNK, False, jnp.float32)
    return _tc_c(z2_parts, u_pad, dinv, b2p, batch2d)


# speedup vs baseline: 29.6883x; 1.0038x over previous
import jax
import jax.numpy as jnp
from jax import lax
from jax.experimental import pallas as pl
from jax.experimental.pallas import tpu as pltpu
from jax.experimental.pallas import tpu_sc as plsc

N = 10000
E = 320000
G = 64
F1 = 128
FP = 16
NC, NS = 2, 16
NW = NC * NS
CHUNK = 128
NCHUNK = 80
EPW = NCHUNK * CHUNK
E_PAD = EPW * NW
FH = F1 // NC
NCHUNK_F = E_PAD // (NS * CHUNK)
NPAD = 10112
RPT = NPAD // NS


def _sc_mesh():
    return plsc.VectorSubcoreMesh(core_axis_name="c", subcore_axis_name="s")


def _deg_body(dst_hbm, ones_hbm, zeros_hbm, out_hbm, acc_sh, idx2_v, rows_v, sem):
    c = lax.axis_index("c")
    s = lax.axis_index("s")
    w = s * NC + c
    pltpu.sync_copy(zeros_hbm, acc_sh.at[pl.ds(s * RPT, RPT)])
    pltpu.sync_copy(ones_hbm, rows_v)
    pltpu.sync_copy(dst_hbm.at[w], idx2_v)
    plsc.subcore_barrier()

    @pl.loop(0, NCHUNK)
    def _fire(i):
        pltpu.async_copy(rows_v, acc_sh.at[idx2_v.at[i]], sem, add=True)

    @pl.loop(0, NCHUNK)
    def _drain(i):
        pltpu.make_async_copy(rows_v, acc_sh.at[idx2_v.at[i]], sem).wait()

    plsc.subcore_barrier()
    pltpu.sync_copy(acc_sh.at[pl.ds(s * RPT, RPT)],
                    out_hbm.at[c, pl.ds(s * RPT, RPT)])


def _sc_degree(dst3, ones_blk, zeros_blk):
    k = pl.kernel(
        _deg_body,
        out_type=jax.ShapeDtypeStruct((NC, NPAD, FP), jnp.bfloat16),
        mesh=_sc_mesh(),
        compiler_params=pltpu.CompilerParams(use_tc_tiling_on_sc=False),
        scratch_types=[
            pltpu.VMEM_SHARED((NPAD, FP), jnp.bfloat16),
            pltpu.VMEM((NCHUNK, CHUNK), jnp.int32),
            pltpu.VMEM((CHUNK, FP), jnp.bfloat16),
            pltpu.SemaphoreType.DMA,
        ],
    )
    return k(dst3, ones_blk, zeros_blk)


def _make_prop_body(featc, ring, ahead, nchunks, fsplit):
    def body(src_hbm, dst_hbm, table_hbm, zeros_hbm, out_hbm,
             acc_sh, src2_v, dst2_v, *rest):
        rows = rest[:ring]
        sg = rest[ring:2 * ring]
        ss = rest[2 * ring:3 * ring]
        c = lax.axis_index("c")
        s = lax.axis_index("s")
        slab = s if fsplit else s * NC + c
        table = table_hbm.at[c] if fsplit else table_hbm
        pltpu.sync_copy(zeros_hbm, acc_sh.at[pl.ds(s * RPT, RPT)])
        pltpu.sync_copy(src_hbm.at[slab], src2_v)
        pltpu.sync_copy(dst_hbm.at[slab], dst2_v)
        plsc.subcore_barrier()

        for g in range(ahead):
            pltpu.async_copy(table.at[src2_v.at[g]], rows[g], sg[g])

        @pl.loop(0, nchunks // ring)
        def _blk(j):
            for b in range(ring):
                g = j * ring + b
                pltpu.make_async_copy(table.at[src2_v.at[g]],
                                      rows[b], sg[b]).wait()
                pltpu.async_copy(rows[b], acc_sh.at[dst2_v.at[g]], ss[b],
                                 add=True)
                g2 = g + ahead
                b2 = (b + ahead) % ring

                @pl.when(g2 < nchunks)
                def _fire_next():
                    @pl.when(g2 >= ring)
                    def _free_buf():
                        pltpu.make_async_copy(
                            rows[b2], acc_sh.at[dst2_v.at[g2 - ring]],
                            ss[b2]).wait()

                    pltpu.async_copy(table.at[src2_v.at[g2]],
                                     rows[b2], sg[b2])

        for b in range(ring):
            g = nchunks - ring + b
            pltpu.make_async_copy(rows[b], acc_sh.at[dst2_v.at[g]],
                                  ss[b]).wait()

        plsc.subcore_barrier()
        pltpu.sync_copy(acc_sh.at[pl.ds(s * RPT, RPT)],
                        out_hbm.at[c, pl.ds(s * RPT, RPT)])

    return body


def _sc_propagate(src3, dst3, table, zeros_blk, featc, ring, ahead, nchunks,
                  fsplit, dtype):
    k = pl.kernel(
        _make_prop_body(featc, ring, ahead, nchunks, fsplit),
        out_type=jax.ShapeDtypeStruct((NC, NPAD, featc), dtype),
        mesh=_sc_mesh(),
        compiler_params=pltpu.CompilerParams(use_tc_tiling_on_sc=False),
        scratch_types=(
            [
                pltpu.VMEM_SHARED((NPAD, featc), dtype),
                pltpu.VMEM((nchunks, CHUNK), jnp.int32),
                pltpu.VMEM((nchunks, CHUNK), jnp.int32),
            ]
            + [pltpu.VMEM((CHUNK, featc), dtype) for _ in range(ring)]
            + [pltpu.SemaphoreType.DMA for _ in range(2 * ring)]
        ),
    )
    return k(src3, dst3, table, zeros_blk)


def _tc_a0_body(x_ref, w1_ref, xw_ref):
    xw_ref[...] = jnp.dot(x_ref[...], w1_ref[...],
                          preferred_element_type=jnp.float32)


def _tc_a0(x, w1):
    return pl.pallas_call(
        _tc_a0_body,
        out_shape=jax.ShapeDtypeStruct((N, F1), jnp.float32),
    )(x, w1)


def _tc_a_body(deg_ref, xw_ref, y_ref, dinv_ref):
    p0 = deg_ref[0, 0:N, 0:1].astype(jnp.float32)
    p1 = deg_ref[1, 0:N, 0:1].astype(jnp.float32)
    dinv = lax.rsqrt(1.0 + p0 + p1)
    xw = xw_ref[...] * dinv
    y_ref[0, 0:N, :] = xw[:, 0:FH].astype(jnp.bfloat16)
    y_ref[1, 0:N, :] = xw[:, FH:F1].astype(jnp.bfloat16)
    y_ref[0, N:NPAD, :] = jnp.zeros((NPAD - N, FH), jnp.bfloat16)
    y_ref[1, N:NPAD, :] = jnp.zeros((NPAD - N, FH), jnp.bfloat16)
    dinv_ref[...] = dinv


def _tc_a(deg_parts, xw):
    return pl.pallas_call(
        _tc_a_body,
        out_shape=[
            jax.ShapeDtypeStruct((NC, NPAD, FH), jnp.bfloat16),
            jax.ShapeDtypeStruct((N, 1), jnp.float32),
        ],
    )(deg_parts, xw)


def _tc_b_body(z_ref, y_ref, dinv_ref, b1_ref, w2_ref, u_ref):
    dinv = dinv_ref[...]
    zy = jnp.concatenate(
        [z_ref[0, 0:N, :].astype(jnp.float32) +
         y_ref[0, 0:N, :].astype(jnp.float32),
         z_ref[1, 0:N, :].astype(jnp.float32) +
         y_ref[1, 0:N, :].astype(jnp.float32)], axis=1)
    h = dinv * zy + b1_ref[...]
    h = jnp.where(h >= 0.0, h, 0.01 * h)
    u = jnp.dot(h, w2_ref[...], preferred_element_type=jnp.float32) * dinv
    u_ref[0:N, :] = u.astype(jnp.bfloat16)
    u_ref[N:NPAD, :] = jnp.zeros((NPAD - N, FP), jnp.bfloat16)


def _tc_b(z_parts, y_pad, dinv, b1, w2p):
    return pl.pallas_call(
        _tc_b_body,
        out_shape=jax.ShapeDtypeStruct((NPAD, FP), jnp.bfloat16),
    )(z_parts, y_pad, dinv, b1, w2p)


def _tc_c_body(z2_ref, u_ref, dinv_ref, b2_ref, batch_ref, out_ref):
    o = dinv_ref[...] * (z2_ref[0, 0:N, :].astype(jnp.float32) +
                         z2_ref[1, 0:N, :].astype(jnp.float32) +
                         u_ref[0:N, :].astype(jnp.float32)) + b2_ref[...]
    col = lax.broadcasted_iota(jnp.int32, (N, FP), 1)
    feat = jnp.where(col == 4, 1.0, o)
    onehot = (batch_ref[...] ==
              lax.broadcasted_iota(jnp.int32, (N, G), 1)).astype(jnp.float32)
    pooled = lax.dot_general(onehot, feat, (((0,), (0,)), ((), ())),
                             preferred_element_type=jnp.float32)
    cnt = jnp.maximum(pooled[:, 4:5], 1.0)
    mean = pooled[:, 0:4] / cnt
    m = jnp.max(mean, axis=1, keepdims=True)
    e = jnp.exp(mean - m)
    out_ref[...] = e / jnp.sum(e, axis=1, keepdims=True)


def _tc_c(z2_parts, u_pad, dinv, b2p, batch2d):
    return pl.pallas_call(
        _tc_c_body,
        out_shape=jax.ShapeDtypeStruct((G, 4), jnp.float32),
    )(z2_parts, u_pad, dinv, b2p, batch2d)


def kernel(x, edge_index, batch, W1, b1, W2, b2):
    ei = edge_index.astype(jnp.int32)
    pad = jnp.full((E_PAD - E,), N, dtype=jnp.int32)
    src_flat = jnp.concatenate([ei[0], pad])
    dst_flat = jnp.concatenate([ei[1], pad])
    src_pad = src_flat.reshape(NW, NCHUNK, CHUNK)
    dst_pad = dst_flat.reshape(NW, NCHUNK, CHUNK)
    src_f = src_flat.reshape(NS, NCHUNK_F, CHUNK)
    dst_f = dst_flat.reshape(NS, NCHUNK_F, CHUNK)
    batch2d = batch.astype(jnp.int32).reshape(N, 1)
    b1r = b1.reshape(1, F1)
    w2p = jnp.zeros((F1, FP), jnp.float32).at[:, 0:4].set(W2)
    b2p = jnp.zeros((1, FP), jnp.float32).at[0, 0:4].set(b2)
    ones_blk = jnp.ones((CHUNK, FP), jnp.bfloat16)
    zeros_fp = jnp.zeros((RPT, FP), jnp.bfloat16)
    zeros_fh = jnp.zeros((RPT, FH), jnp.bfloat16)

    xw = _tc_a0(x, W1)
    deg_parts = _sc_degree(dst_pad, ones_blk, zeros_fp)
    y2, dinv = _tc_a(deg_parts, xw)
    z_parts = _sc_propagate(src_f, dst_f, y2, zeros_fh, FH, 5, 3, NCHUNK_F,
                            True, jnp.bfloat16)
    u_pad = _tc_b(z_parts, y2, dinv, b1r, w2p)
    z2_parts = _sc_propagate(src_pad, dst_pad, u_pad, zeros_fp, FP, 10, 5,
                             NCHUNK, False, jnp.bfloat16)
    return _tc_c(z2_parts, u_pad, dinv, b2p, batch2d)
